# baseline (device time: 95266 ns/iter reference)
import os

import jax
import jax.numpy as jnp
from jax import lax
from jax.experimental import pallas as pl
from jax.experimental.pallas import tpu as pltpu

N_DEV = 4
B = 2
SQ = 512
SKV = 512
HQ = 8
DH = 64
DM = 768
DQ = HQ * DH
BLK = 64
CHUNK = SQ // N_DEV
BF = jnp.bfloat16
NO_COMM = bool(int(os.environ.get("KERNEL_NO_COMM", "0")))


def kernel(x, Wq, K_ext, V_ext, Wo):
    my_pos = lax.axis_index("i")
    K = lax.dynamic_slice_in_dim(K_ext, my_pos * HQ, HQ, axis=2).astype(BF)
    V = lax.dynamic_slice_in_dim(V_ext, my_pos * HQ, HQ, axis=2).astype(BF)

    def body(x_ref, wq_ref, k_ref, v_ref, wo_ref, out_ref,
             work_ref, comm_ref, wq_bf, wo_bf, send_sems, recv_sems):
        pos = lax.axis_index("i")
        left = lax.rem(pos + N_DEV - 1, N_DEV)
        right = lax.rem(pos + 1, N_DEV)

        barrier = pltpu.get_barrier_semaphore()
        for nbr in (left, right):
            pl.semaphore_signal(
                barrier, inc=1,
                device_id=(nbr,), device_id_type=pl.DeviceIdType.MESH,
            )
        pl.semaphore_wait(barrier, 2)

        wq_bf[...] = (wq_ref[...] * 0.125).astype(BF)
        wo_bf[...] = wo_ref[...].astype(BF)

        def compute_chunk(c):
            r0 = c * CHUNK
            kvl = (c + 1) * CHUNK
            row_blk = (
                lax.broadcasted_iota(jnp.int32, (CHUNK, kvl), 0) + r0
            ) // BLK
            col_blk = lax.broadcasted_iota(jnp.int32, (CHUNK, kvl), 1) // BLK
            bias = jnp.where(col_blk <= row_blk, 0.0, -1e9).astype(jnp.float32)
            xc2 = jnp.concatenate(
                [x_ref[b, r0:r0 + CHUNK, :] for b in range(B)], axis=0
            ).astype(BF)
            q2 = jnp.dot(xc2, wq_bf[...],
                         preferred_element_type=jnp.float32
                         ).astype(BF)
            ctx_rows = []
            for b in range(B):
                ctxs = []
                for h in range(HQ):
                    qh = q2[b * CHUNK:(b + 1) * CHUNK, h * DH:(h + 1) * DH]
                    s = lax.dot_general(
                        qh, k_ref[b, 0:kvl, h, :], (((1,), (1,)), ((), ())),
                        preferred_element_type=jnp.float32,
                    ) + bias
                    w = jnp.exp(s)
                    rs = jnp.sum(w, axis=-1, keepdims=True)
                    ctx = lax.dot_general(
                        w.astype(BF), v_ref[b, 0:kvl, h, :],
                        (((1,), (0,)), ((), ())),
                        preferred_element_type=jnp.float32,
                    ) * (1.0 / rs)
                    ctxs.append(ctx.astype(BF))
                ctx_rows.append(jnp.concatenate(ctxs, axis=1))
            ctx2 = jnp.concatenate(ctx_rows, axis=0)
            o2 = jnp.dot(ctx2, wo_bf[...],
                         preferred_element_type=jnp.float32
                         ).astype(BF)
            work_ref[0, r0:r0 + CHUNK, :] = o2[0:CHUNK]
            work_ref[1, r0:r0 + CHUNK, :] = o2[CHUNK:2 * CHUNK]

        def chunk_at(b, c):
            return work_ref.at[b, pl.ds(c * CHUNK, CHUNK), :]

        def rs_pair(hop, c_cw, c_ccw):
            cw = pltpu.make_async_remote_copy(
                src_ref=chunk_at(0, c_cw),
                dst_ref=comm_ref.at[hop, 0],
                send_sem=send_sems.at[2 * hop],
                recv_sem=recv_sems.at[2 * hop],
                device_id=(right,), device_id_type=pl.DeviceIdType.MESH,
            )
            ccw = pltpu.make_async_remote_copy(
                src_ref=chunk_at(1, c_ccw),
                dst_ref=comm_ref.at[hop, 1],
                send_sem=send_sems.at[2 * hop + 1],
                recv_sem=recv_sems.at[2 * hop + 1],
                device_id=(left,), device_id_type=pl.DeviceIdType.MESH,
            )
            return cw, ccw

        def rs_finish(cw, ccw, hop, c_cw_recv, c_ccw_recv, recv_only=False):
            if recv_only:
                cw.wait_recv()
                ccw.wait_recv()
            else:
                cw.wait()
                ccw.wait()
            r0 = c_cw_recv * CHUNK
            work_ref[0, pl.ds(r0, CHUNK), :] = (
                work_ref[0, pl.ds(r0, CHUNK), :] + comm_ref[hop, 0]
            )
            r1 = c_ccw_recv * CHUNK
            work_ref[1, pl.ds(r1, CHUNK), :] = (
                work_ref[1, pl.ds(r1, CHUNK), :] + comm_ref[hop, 1]
            )

        c_m1 = lax.rem(pos + 3, N_DEV)
        c_p1 = lax.rem(pos + 1, N_DEV)
        c_p2 = lax.rem(pos + 2, N_DEV)

        if NO_COMM:
            for c in range(N_DEV):
                compute_chunk(c)
            out_ref[...] = work_ref[...].astype(jnp.float32)
            return

        h0cw, h0ccw = rs_pair(0, pos, pos)
        h1cw, h1ccw = rs_pair(1, c_m1, c_p1)
        h2cw, h2ccw = rs_pair(2, c_p2, c_p2)

        def compute_slot(rel):
            for c in range(N_DEV):
                @pl.when(pos == (c - rel) % N_DEV)
                def _():
                    compute_chunk(c)

        compute_slot(0)
        h0cw.start()
        h0ccw.start()
        compute_slot(-1)
        compute_slot(1)
        rs_finish(h0cw, h0ccw, 0, c_m1, c_p1, recv_only=True)
        h1cw.start()
        h1ccw.start()
        compute_slot(2)
        h0cw.wait_send()
        h0ccw.wait_send()
        rs_finish(h1cw, h1ccw, 1, c_p2, c_p2)

        h2cw.start()
        h2ccw.start()
        rs_finish(h2cw, h2ccw, 2, c_p1, c_m1)

        for g in range(N_DEV - 1):
            c_cw = lax.rem(pos + N_DEV + 1 - g, N_DEV)
            c_ccw = lax.rem(pos + N_DEV - 1 + g, N_DEV)
            k = 2 * (N_DEV - 1 + g)
            cw = pltpu.make_async_remote_copy(
                src_ref=chunk_at(0, c_cw), dst_ref=chunk_at(0, c_cw),
                send_sem=send_sems.at[k], recv_sem=recv_sems.at[k],
                device_id=(right,), device_id_type=pl.DeviceIdType.MESH,
            )
            ccw = pltpu.make_async_remote_copy(
                src_ref=chunk_at(1, c_ccw), dst_ref=chunk_at(1, c_ccw),
                send_sem=send_sems.at[k + 1], recv_sem=recv_sems.at[k + 1],
                device_id=(left,), device_id_type=pl.DeviceIdType.MESH,
            )
            cw.start()
            ccw.start()
            cw.wait()
            ccw.wait()

        out_ref[...] = work_ref[...].astype(jnp.float32)

    return pl.pallas_call(
        body,
        out_shape=jax.ShapeDtypeStruct((B, SQ, DM), jnp.float32),
        in_specs=[pl.BlockSpec(memory_space=pltpu.VMEM)] * 5,
        out_specs=pl.BlockSpec(memory_space=pltpu.VMEM),
        scratch_shapes=[
            pltpu.VMEM((B, SQ, DM), BF),
            pltpu.VMEM((N_DEV - 1, 2, CHUNK, DM), BF),
            pltpu.VMEM((DM, DQ), BF),
            pltpu.VMEM((DQ, DM), BF),
            pltpu.SemaphoreType.DMA((4 * (N_DEV - 1),)),
            pltpu.SemaphoreType.DMA((4 * (N_DEV - 1),)),
        ],
        compiler_params=pltpu.CompilerParams(collective_id=0),
    )(x, Wq, K, V, Wo)


# device time: 47147 ns/iter; 2.0206x vs baseline; 2.0206x over previous
import os

import jax
import jax.numpy as jnp
from jax import lax
from jax.experimental import pallas as pl
from jax.experimental.pallas import tpu as pltpu

N_DEV = 4
B = 2
SQ = 512
SKV = 512
HQ = 8
DH = 64
DM = 768
DQ = HQ * DH
BLK = 64
CHUNK = SQ // N_DEV
BF = jnp.bfloat16
NO_COMM = bool(int(os.environ.get("KERNEL_NO_COMM", "0")))


def kernel(x, Wq, K_ext, V_ext, Wo):
    my_pos = lax.axis_index("i")
    K = jnp.transpose(
        lax.dynamic_slice_in_dim(K_ext, my_pos * HQ, HQ, axis=2), (0, 2, 1, 3)
    ).astype(BF)
    V = jnp.transpose(
        lax.dynamic_slice_in_dim(V_ext, my_pos * HQ, HQ, axis=2), (0, 2, 1, 3)
    ).astype(BF)

    def body(x_ref, wq_ref, k_ref, v_ref, wo_ref, out_ref,
             work_ref, comm_ref, wq_bf, wo_bf, send_sems, recv_sems):
        pos = lax.axis_index("i")
        left = lax.rem(pos + N_DEV - 1, N_DEV)
        right = lax.rem(pos + 1, N_DEV)

        barrier = pltpu.get_barrier_semaphore()
        for nbr in (left, right):
            pl.semaphore_signal(
                barrier, inc=1,
                device_id=(nbr,), device_id_type=pl.DeviceIdType.MESH,
            )
        pl.semaphore_wait(barrier, 2)

        wq_bf[...] = (wq_ref[...] * 0.125).astype(BF)
        wo_bf[...] = wo_ref[...].astype(BF)

        def compute_chunk(c):
            r0 = c * CHUNK
            kvl = (c + 1) * CHUNK
            row_blk = (
                lax.broadcasted_iota(jnp.int32, (CHUNK, kvl), 0) + r0
            ) // BLK
            col_blk = lax.broadcasted_iota(jnp.int32, (CHUNK, kvl), 1) // BLK
            bias = jnp.where(col_blk <= row_blk, 0.0, -1e9).astype(jnp.float32)
            xc2 = jnp.concatenate(
                [x_ref[b, r0:r0 + CHUNK, :] for b in range(B)], axis=0
            ).astype(BF)
            q2 = jnp.dot(xc2, wq_bf[...],
                         preferred_element_type=jnp.float32
                         ).astype(BF)
            ctx_rows = []
            for b in range(B):
                ctxs = []
                for h in range(HQ):
                    qh = q2[b * CHUNK:(b + 1) * CHUNK, h * DH:(h + 1) * DH]
                    s = lax.dot_general(
                        qh, k_ref[b, h, 0:kvl, :], (((1,), (1,)), ((), ())),
                        preferred_element_type=jnp.float32,
                    ) + bias
                    w = jnp.exp(s)
                    rs = jnp.sum(w, axis=-1, keepdims=True)
                    ctx = lax.dot_general(
                        w.astype(BF), v_ref[b, h, 0:kvl, :],
                        (((1,), (0,)), ((), ())),
                        preferred_element_type=jnp.float32,
                    ) * (1.0 / rs)
                    ctxs.append(ctx.astype(BF))
                ctx_rows.append(jnp.concatenate(ctxs, axis=1))
            ctx2 = jnp.concatenate(ctx_rows, axis=0)
            o2 = jnp.dot(ctx2, wo_bf[...],
                         preferred_element_type=jnp.float32
                         ).astype(BF)
            work_ref[0, r0:r0 + CHUNK, :] = o2[0:CHUNK]
            work_ref[1, r0:r0 + CHUNK, :] = o2[CHUNK:2 * CHUNK]

        def chunk_at(b, c):
            return work_ref.at[b, pl.ds(c * CHUNK, CHUNK), :]

        def rs_pair(hop, c_cw, c_ccw):
            cw = pltpu.make_async_remote_copy(
                src_ref=chunk_at(0, c_cw),
                dst_ref=comm_ref.at[hop, 0],
                send_sem=send_sems.at[2 * hop],
                recv_sem=recv_sems.at[2 * hop],
                device_id=(right,), device_id_type=pl.DeviceIdType.MESH,
            )
            ccw = pltpu.make_async_remote_copy(
                src_ref=chunk_at(1, c_ccw),
                dst_ref=comm_ref.at[hop, 1],
                send_sem=send_sems.at[2 * hop + 1],
                recv_sem=recv_sems.at[2 * hop + 1],
                device_id=(left,), device_id_type=pl.DeviceIdType.MESH,
            )
            return cw, ccw

        def rs_finish(cw, ccw, hop, c_cw_recv, c_ccw_recv, recv_only=False):
            if recv_only:
                cw.wait_recv()
                ccw.wait_recv()
            else:
                cw.wait()
                ccw.wait()
            r0 = c_cw_recv * CHUNK
            work_ref[0, pl.ds(r0, CHUNK), :] = (
                work_ref[0, pl.ds(r0, CHUNK), :] + comm_ref[hop, 0]
            )
            r1 = c_ccw_recv * CHUNK
            work_ref[1, pl.ds(r1, CHUNK), :] = (
                work_ref[1, pl.ds(r1, CHUNK), :] + comm_ref[hop, 1]
            )

        c_m1 = lax.rem(pos + 3, N_DEV)
        c_p1 = lax.rem(pos + 1, N_DEV)
        c_p2 = lax.rem(pos + 2, N_DEV)

        if NO_COMM:
            for c in range(N_DEV):
                compute_chunk(c)
            out_ref[...] = work_ref[...].astype(jnp.float32)
            return

        h0cw, h0ccw = rs_pair(0, pos, pos)
        h1cw, h1ccw = rs_pair(1, c_m1, c_p1)
        h2cw, h2ccw = rs_pair(2, c_p2, c_p2)

        def compute_slot(rel):
            for c in range(N_DEV):
                @pl.when(pos == (c - rel) % N_DEV)
                def _():
                    compute_chunk(c)

        compute_slot(0)
        h0cw.start()
        h0ccw.start()
        compute_slot(-1)
        compute_slot(1)
        rs_finish(h0cw, h0ccw, 0, c_m1, c_p1, recv_only=True)
        h1cw.start()
        h1ccw.start()
        compute_slot(2)
        h0cw.wait_send()
        h0ccw.wait_send()
        rs_finish(h1cw, h1ccw, 1, c_p2, c_p2)

        h2cw.start()
        h2ccw.start()
        rs_finish(h2cw, h2ccw, 2, c_p1, c_m1)

        for g in range(N_DEV - 1):
            c_cw = lax.rem(pos + N_DEV + 1 - g, N_DEV)
            c_ccw = lax.rem(pos + N_DEV - 1 + g, N_DEV)
            k = 2 * (N_DEV - 1 + g)
            cw = pltpu.make_async_remote_copy(
                src_ref=chunk_at(0, c_cw), dst_ref=chunk_at(0, c_cw),
                send_sem=send_sems.at[k], recv_sem=recv_sems.at[k],
                device_id=(right,), device_id_type=pl.DeviceIdType.MESH,
            )
            ccw = pltpu.make_async_remote_copy(
                src_ref=chunk_at(1, c_ccw), dst_ref=chunk_at(1, c_ccw),
                send_sem=send_sems.at[k + 1], recv_sem=recv_sems.at[k + 1],
                device_id=(left,), device_id_type=pl.DeviceIdType.MESH,
            )
            cw.start()
            ccw.start()
            cw.wait()
            ccw.wait()

        out_ref[...] = work_ref[...].astype(jnp.float32)

    return pl.pallas_call(
        body,
        out_shape=jax.ShapeDtypeStruct((B, SQ, DM), jnp.float32),
        in_specs=[pl.BlockSpec(memory_space=pltpu.VMEM)] * 5,
        out_specs=pl.BlockSpec(memory_space=pltpu.VMEM),
        scratch_shapes=[
            pltpu.VMEM((B, SQ, DM), BF),
            pltpu.VMEM((N_DEV - 1, 2, CHUNK, DM), BF),
            pltpu.VMEM((DM, DQ), BF),
            pltpu.VMEM((DQ, DM), BF),
            pltpu.SemaphoreType.DMA((4 * (N_DEV - 1),)),
            pltpu.SemaphoreType.DMA((4 * (N_DEV - 1),)),
        ],
        compiler_params=pltpu.CompilerParams(collective_id=0),
    )(x, Wq, K, V, Wo)


# device time: 44114 ns/iter; 2.1595x vs baseline; 1.0688x over previous
import os

import jax
import jax.numpy as jnp
from jax import lax
from jax.experimental import pallas as pl
from jax.experimental.pallas import tpu as pltpu

N_DEV = 4
B = 2
SQ = 512
SKV = 512
HQ = 8
DH = 64
DM = 768
DQ = HQ * DH
BLK = 64
CHUNK = SQ // N_DEV
BF = jnp.bfloat16
NO_COMM = bool(int(os.environ.get("KERNEL_NO_COMM", "0")))


def kernel(x, Wq, K_ext, V_ext, Wo):
    my_pos = lax.axis_index("i")
    K = jnp.transpose(
        lax.dynamic_slice_in_dim(K_ext, my_pos * HQ, HQ, axis=2), (0, 2, 1, 3)
    ).astype(BF)
    V = jnp.transpose(
        lax.dynamic_slice_in_dim(V_ext, my_pos * HQ, HQ, axis=2), (0, 2, 1, 3)
    ).astype(BF)

    def body(x_ref, wq_ref, k_ref, v_ref, wo_ref, out_ref,
             work_ref, comm_ref, wq_bf, wo_bf, send_sems, recv_sems):
        pos = lax.axis_index("i")
        left = lax.rem(pos + N_DEV - 1, N_DEV)
        right = lax.rem(pos + 1, N_DEV)

        barrier = pltpu.get_barrier_semaphore()
        for nbr in (left, right):
            pl.semaphore_signal(
                barrier, inc=1,
                device_id=(nbr,), device_id_type=pl.DeviceIdType.MESH,
            )
        pl.semaphore_wait(barrier, 2)

        wq_bf[...] = (wq_ref[...] * 0.125).astype(BF)
        wo_bf[...] = wo_ref[...].astype(BF)

        def compute_chunk(c):
            r0 = c * CHUNK
            kvl = (c + 1) * CHUNK
            row_blk = (
                lax.broadcasted_iota(jnp.int32, (CHUNK, kvl), 0) + r0
            ) // BLK
            col_blk = lax.broadcasted_iota(jnp.int32, (CHUNK, kvl), 1) // BLK
            bias = jnp.where(col_blk <= row_blk, 0.0, -1e9).astype(jnp.float32)
            xc2 = jnp.concatenate(
                [x_ref[b, r0:r0 + CHUNK, :] for b in range(B)], axis=0
            ).astype(BF)
            q2 = jnp.dot(xc2, wq_bf[...],
                         preferred_element_type=jnp.float32
                         ).astype(BF)
            ctx_rows = []
            for b in range(B):
                ctxs = []
                for h in range(HQ):
                    qh = q2[b * CHUNK:(b + 1) * CHUNK, h * DH:(h + 1) * DH]
                    s = lax.dot_general(
                        qh, k_ref[b, h, 0:kvl, :], (((1,), (1,)), ((), ())),
                        preferred_element_type=jnp.float32,
                    ) + bias
                    w = jnp.exp(s)
                    rs = jnp.sum(w, axis=-1, keepdims=True)
                    ctx = lax.dot_general(
                        w.astype(BF), v_ref[b, h, 0:kvl, :],
                        (((1,), (0,)), ((), ())),
                        preferred_element_type=jnp.float32,
                    ) * (1.0 / rs)
                    ctxs.append(ctx.astype(BF))
                ctx_rows.append(jnp.concatenate(ctxs, axis=1))
            ctx2 = jnp.concatenate(ctx_rows, axis=0)
            o2 = jnp.dot(ctx2, wo_bf[...],
                         preferred_element_type=jnp.float32
                         ).astype(BF)
            work_ref[0, r0:r0 + CHUNK, :] = o2[0:CHUNK]
            work_ref[1, r0:r0 + CHUNK, :] = o2[CHUNK:2 * CHUNK]

        def chunk_at(b, c):
            return work_ref.at[b, pl.ds(c * CHUNK, CHUNK), :]

        def rs_pair(hop, c_cw, c_ccw):
            cw = pltpu.make_async_remote_copy(
                src_ref=chunk_at(0, c_cw),
                dst_ref=comm_ref.at[hop, 0],
                send_sem=send_sems.at[2 * hop],
                recv_sem=recv_sems.at[2 * hop],
                device_id=(right,), device_id_type=pl.DeviceIdType.MESH,
            )
            ccw = pltpu.make_async_remote_copy(
                src_ref=chunk_at(1, c_ccw),
                dst_ref=comm_ref.at[hop, 1],
                send_sem=send_sems.at[2 * hop + 1],
                recv_sem=recv_sems.at[2 * hop + 1],
                device_id=(left,), device_id_type=pl.DeviceIdType.MESH,
            )
            return cw, ccw

        def rs_finish(cw, ccw, hop, c_cw_recv, c_ccw_recv, recv_only=False):
            if recv_only:
                cw.wait_recv()
                ccw.wait_recv()
            else:
                cw.wait()
                ccw.wait()
            r0 = c_cw_recv * CHUNK
            work_ref[0, pl.ds(r0, CHUNK), :] = (
                work_ref[0, pl.ds(r0, CHUNK), :] + comm_ref[hop, 0]
            )
            r1 = c_ccw_recv * CHUNK
            work_ref[1, pl.ds(r1, CHUNK), :] = (
                work_ref[1, pl.ds(r1, CHUNK), :] + comm_ref[hop, 1]
            )

        c_m1 = lax.rem(pos + 3, N_DEV)
        c_p1 = lax.rem(pos + 1, N_DEV)
        c_p2 = lax.rem(pos + 2, N_DEV)

        if NO_COMM:
            for c in range(N_DEV):
                compute_chunk(c)
            out_ref[...] = work_ref[...].astype(jnp.float32)
            return

        h0cw, h0ccw = rs_pair(0, pos, pos)
        h1cw, h1ccw = rs_pair(1, c_m1, c_p1)

        def compute_slot(rel):
            for c in range(N_DEV):
                @pl.when(pos == (c - rel) % N_DEV)
                def _():
                    compute_chunk(c)

        compute_slot(0)
        h0cw.start()
        h0ccw.start()
        compute_slot(-1)
        compute_slot(1)
        rs_finish(h0cw, h0ccw, 0, c_m1, c_p1, recv_only=True)
        h1cw.start()
        h1ccw.start()
        compute_slot(2)
        h0cw.wait_send()
        h0ccw.wait_send()
        rs_finish(h1cw, h1ccw, 1, c_p2, c_p2)

        HALF = CHUNK // 2

        def half_at(b, c, half):
            return work_ref.at[b, pl.ds(c * CHUNK + half * HALF, HALF), :]

        dirs = ((0, c_p2, right), (1, c_p2, left))
        h2 = {}
        for d, (bidx, c_src, tgt) in enumerate(dirs):
            for half in range(2):
                h2[(d, half)] = pltpu.make_async_remote_copy(
                    src_ref=half_at(bidx, c_src, half),
                    dst_ref=comm_ref.at[2, d, pl.ds(half * HALF, HALF), :],
                    send_sem=send_sems.at[4 + 2 * d + half],
                    recv_sem=recv_sems.at[4 + 2 * d + half],
                    device_id=(tgt,), device_id_type=pl.DeviceIdType.MESH,
                )
        for half in range(2):
            h2[(0, half)].start()
            h2[(1, half)].start()

        agd = []
        for g in range(N_DEV - 1):
            c_cw = lax.rem(pos + N_DEV + 1 - g, N_DEV)
            c_ccw = lax.rem(pos + N_DEV - 1 + g, N_DEV)
            per_dir = []
            for d, (c_src, tgt) in enumerate(((c_cw, right), (c_ccw, left))):
                per_dir.append([
                    pltpu.make_async_remote_copy(
                        src_ref=half_at(d, c_src, half),
                        dst_ref=half_at(d, c_src, half),
                        send_sem=send_sems.at[8 + 4 * g + 2 * d + half],
                        recv_sem=recv_sems.at[8 + 4 * g + 2 * d + half],
                        device_id=(tgt,), device_id_type=pl.DeviceIdType.MESH,
                    )
                    for half in range(2)
                ])
            agd.append(per_dir)

        add_tgt = (c_p1, c_m1)
        for half in range(2):
            for d in range(2):
                h2[(d, half)].wait_recv()
                r = add_tgt[d] * CHUNK + half * HALF
                work_ref[d, pl.ds(r, HALF), :] = (
                    work_ref[d, pl.ds(r, HALF), :]
                    + comm_ref[2, d, pl.ds(half * HALF, HALF), :]
                )
                agd[0][d][half].start()

        for g in range(1, N_DEV - 1):
            for half in range(2):
                for d in range(2):
                    agd[g - 1][d][half].wait_recv()
                    agd[g][d][half].start()
        for half in range(2):
            for d in range(2):
                agd[N_DEV - 2][d][half].wait_recv()

        for rdma in h2.values():
            rdma.wait_send()
        for g in range(N_DEV - 1):
            for d in range(2):
                for half in range(2):
                    agd[g][d][half].wait_send()

        out_ref[...] = work_ref[...].astype(jnp.float32)

    return pl.pallas_call(
        body,
        out_shape=jax.ShapeDtypeStruct((B, SQ, DM), jnp.float32),
        in_specs=[pl.BlockSpec(memory_space=pltpu.VMEM)] * 5,
        out_specs=pl.BlockSpec(memory_space=pltpu.VMEM),
        scratch_shapes=[
            pltpu.VMEM((B, SQ, DM), BF),
            pltpu.VMEM((N_DEV - 1, 2, CHUNK, DM), BF),
            pltpu.VMEM((DM, DQ), BF),
            pltpu.VMEM((DQ, DM), BF),
            pltpu.SemaphoreType.DMA((20,)),
            pltpu.SemaphoreType.DMA((20,)),
        ],
        compiler_params=pltpu.CompilerParams(collective_id=0),
    )(x, Wq, K, V, Wo)


# device time: 42749 ns/iter; 2.2285x vs baseline; 1.0319x over previous
import os

import jax
import jax.numpy as jnp
from jax import lax
from jax.experimental import pallas as pl
from jax.experimental.pallas import tpu as pltpu

N_DEV = 4
B = 2
SQ = 512
SKV = 512
HQ = 8
DH = 64
DM = 768
DQ = HQ * DH
BLK = 64
CHUNK = SQ // N_DEV
BF = jnp.bfloat16
NO_COMM = bool(int(os.environ.get("KERNEL_NO_COMM", "0")))


def kernel(x, Wq, K_ext, V_ext, Wo):
    my_pos = lax.axis_index("i")
    K = jnp.transpose(
        lax.dynamic_slice_in_dim(K_ext, my_pos * HQ, HQ, axis=2), (0, 2, 1, 3)
    ).astype(BF)
    V = jnp.transpose(
        lax.dynamic_slice_in_dim(V_ext, my_pos * HQ, HQ, axis=2), (0, 2, 1, 3)
    ).astype(BF)

    def body(x_ref, wq_ref, k_ref, v_ref, wo_ref, out_ref,
             work_ref, comm_ref, wq_bf, wo_bf, send_sems, recv_sems):
        pos = lax.axis_index("i")
        left = lax.rem(pos + N_DEV - 1, N_DEV)
        right = lax.rem(pos + 1, N_DEV)

        barrier = pltpu.get_barrier_semaphore()
        for nbr in (left, right):
            pl.semaphore_signal(
                barrier, inc=1,
                device_id=(nbr,), device_id_type=pl.DeviceIdType.MESH,
            )
        pl.semaphore_wait(barrier, 2)

        wq_bf[...] = (wq_ref[...] * 0.125).astype(BF)
        wo_bf[...] = wo_ref[...].astype(BF)

        def compute_chunk(c):
            r0 = c * CHUNK
            kvl = (c + 1) * CHUNK
            row_blk = (
                lax.broadcasted_iota(jnp.int32, (CHUNK, kvl), 0) + r0
            ) // BLK
            col_blk = lax.broadcasted_iota(jnp.int32, (CHUNK, kvl), 1) // BLK
            bias = jnp.where(col_blk <= row_blk, 0.0, -1e9).astype(jnp.float32)
            xc2 = jnp.concatenate(
                [x_ref[b, r0:r0 + CHUNK, :] for b in range(B)], axis=0
            ).astype(BF)
            q2 = jnp.dot(xc2, wq_bf[...],
                         preferred_element_type=jnp.float32
                         ).astype(BF)
            ctx_rows = []
            for b in range(B):
                ctxs = []
                for h in range(HQ):
                    qh = q2[b * CHUNK:(b + 1) * CHUNK, h * DH:(h + 1) * DH]
                    s = lax.dot_general(
                        qh, k_ref[b, h, 0:kvl, :], (((1,), (1,)), ((), ())),
                        preferred_element_type=jnp.float32,
                    ) + bias
                    w = jnp.exp(s)
                    rs = jnp.sum(w, axis=-1, keepdims=True)
                    ctx = lax.dot_general(
                        w.astype(BF), v_ref[b, h, 0:kvl, :],
                        (((1,), (0,)), ((), ())),
                        preferred_element_type=jnp.float32,
                    ) * (1.0 / rs)
                    ctxs.append(ctx.astype(BF))
                ctx_rows.append(jnp.concatenate(ctxs, axis=1))
            ctx2 = jnp.concatenate(ctx_rows, axis=0)
            o2 = jnp.dot(ctx2, wo_bf[...],
                         preferred_element_type=jnp.float32
                         ).astype(BF)
            work_ref[0, r0:r0 + CHUNK, :] = o2[0:CHUNK]
            work_ref[1, r0:r0 + CHUNK, :] = o2[CHUNK:2 * CHUNK]

        def chunk_at(b, c):
            return work_ref.at[b, pl.ds(c * CHUNK, CHUNK), :]

        def rs_pair(hop, c_cw, c_ccw):
            cw = pltpu.make_async_remote_copy(
                src_ref=chunk_at(0, c_cw),
                dst_ref=comm_ref.at[hop, 0],
                send_sem=send_sems.at[2 * hop],
                recv_sem=recv_sems.at[2 * hop],
                device_id=(right,), device_id_type=pl.DeviceIdType.MESH,
            )
            ccw = pltpu.make_async_remote_copy(
                src_ref=chunk_at(1, c_ccw),
                dst_ref=comm_ref.at[hop, 1],
                send_sem=send_sems.at[2 * hop + 1],
                recv_sem=recv_sems.at[2 * hop + 1],
                device_id=(left,), device_id_type=pl.DeviceIdType.MESH,
            )
            return cw, ccw

        def rs_finish(cw, ccw, hop, c_cw_recv, c_ccw_recv, recv_only=False):
            if recv_only:
                cw.wait_recv()
                ccw.wait_recv()
            else:
                cw.wait()
                ccw.wait()
            r0 = c_cw_recv * CHUNK
            work_ref[0, pl.ds(r0, CHUNK), :] = (
                work_ref[0, pl.ds(r0, CHUNK), :] + comm_ref[hop, 0]
            )
            r1 = c_ccw_recv * CHUNK
            work_ref[1, pl.ds(r1, CHUNK), :] = (
                work_ref[1, pl.ds(r1, CHUNK), :] + comm_ref[hop, 1]
            )

        c_m1 = lax.rem(pos + 3, N_DEV)
        c_p1 = lax.rem(pos + 1, N_DEV)
        c_p2 = lax.rem(pos + 2, N_DEV)

        if NO_COMM:
            for c in range(N_DEV):
                compute_chunk(c)
            out_ref[...] = work_ref[...].astype(jnp.float32)
            return

        h0cw, h0ccw = rs_pair(0, pos, pos)
        h1cw, h1ccw = rs_pair(1, c_m1, c_p1)

        def compute_slot(rel):
            for c in range(N_DEV):
                @pl.when(pos == (c - rel) % N_DEV)
                def _():
                    compute_chunk(c)

        compute_slot(0)
        h0cw.start()
        h0ccw.start()
        compute_slot(-1)
        compute_slot(1)
        rs_finish(h0cw, h0ccw, 0, c_m1, c_p1, recv_only=True)
        h1cw.start()
        h1ccw.start()
        compute_slot(2)
        h0cw.wait_send()
        h0ccw.wait_send()
        rs_finish(h1cw, h1ccw, 1, c_p2, c_p2)

        HALF = CHUNK // 2

        def half_at(b, c, half):
            return work_ref.at[b, pl.ds(c * CHUNK + half * HALF, HALF), :]

        dirs = ((0, c_p2, right), (1, c_p2, left))
        h2 = {}
        for d, (bidx, c_src, tgt) in enumerate(dirs):
            for half in range(2):
                h2[(d, half)] = pltpu.make_async_remote_copy(
                    src_ref=half_at(bidx, c_src, half),
                    dst_ref=comm_ref.at[2, d, pl.ds(half * HALF, HALF), :],
                    send_sem=send_sems.at[4 + 2 * d + half],
                    recv_sem=recv_sems.at[4 + 2 * d + half],
                    device_id=(tgt,), device_id_type=pl.DeviceIdType.MESH,
                )
        for half in range(2):
            h2[(0, half)].start()
            h2[(1, half)].start()

        agd = []
        for g in range(N_DEV - 1):
            c_cw = lax.rem(pos + N_DEV + 1 - g, N_DEV)
            c_ccw = lax.rem(pos + N_DEV - 1 + g, N_DEV)
            per_dir = []
            for d, (c_src, tgt) in enumerate(((c_cw, right), (c_ccw, left))):
                per_dir.append([
                    pltpu.make_async_remote_copy(
                        src_ref=half_at(d, c_src, half),
                        dst_ref=half_at(d, c_src, half),
                        send_sem=send_sems.at[8 + 4 * g + 2 * d + half],
                        recv_sem=recv_sems.at[8 + 4 * g + 2 * d + half],
                        device_id=(tgt,), device_id_type=pl.DeviceIdType.MESH,
                    )
                    for half in range(2)
                ])
            agd.append(per_dir)

        def conv(d, c, half):
            r = c * CHUNK + half * HALF
            out_ref[d, pl.ds(r, HALF), :] = (
                work_ref[d, pl.ds(r, HALF), :].astype(jnp.float32)
            )

        add_tgt = (c_p1, c_m1)
        for half in range(2):
            for d in range(2):
                h2[(d, half)].wait_recv()
                r = add_tgt[d] * CHUNK + half * HALF
                work_ref[d, pl.ds(r, HALF), :] = (
                    work_ref[d, pl.ds(r, HALF), :]
                    + comm_ref[2, d, pl.ds(half * HALF, HALF), :]
                )
                agd[0][d][half].start()
        for half in range(2):
            for d in range(2):
                conv(d, add_tgt[d], half)

        for g in range(1, N_DEV - 1):
            c_arr = (lax.rem(pos + N_DEV - (g - 1), N_DEV),
                     lax.rem(pos + g - 1, N_DEV))
            for half in range(2):
                for d in range(2):
                    agd[g - 1][d][half].wait_recv()
                    agd[g][d][half].start()
                for d in range(2):
                    conv(d, c_arr[d], half)
        c_last = (lax.rem(pos + N_DEV - (N_DEV - 2), N_DEV),
                  lax.rem(pos + N_DEV - 2, N_DEV))
        for half in range(2):
            for d in range(2):
                agd[N_DEV - 2][d][half].wait_recv()
                conv(d, c_last[d], half)

        for rdma in h2.values():
            rdma.wait_send()
        for g in range(N_DEV - 1):
            for d in range(2):
                for half in range(2):
                    agd[g][d][half].wait_send()

    return pl.pallas_call(
        body,
        out_shape=jax.ShapeDtypeStruct((B, SQ, DM), jnp.float32),
        in_specs=[pl.BlockSpec(memory_space=pltpu.VMEM)] * 5,
        out_specs=pl.BlockSpec(memory_space=pltpu.VMEM),
        scratch_shapes=[
            pltpu.VMEM((B, SQ, DM), BF),
            pltpu.VMEM((N_DEV - 1, 2, CHUNK, DM), BF),
            pltpu.VMEM((DM, DQ), BF),
            pltpu.VMEM((DQ, DM), BF),
            pltpu.SemaphoreType.DMA((20,)),
            pltpu.SemaphoreType.DMA((20,)),
        ],
        compiler_params=pltpu.CompilerParams(collective_id=0),
    )(x, Wq, K, V, Wo)
